# Initial kernel scaffold; baseline (speedup 1.0000x reference)
import jax
import jax.numpy as jnp
from jax.experimental import pallas as pl


def kernel(ids, feats, table, w1, b1, w2, b2, wfc, bfc, wf, bf):
    raise NotImplementedError("write your pallas kernel here")


# speedup vs baseline: 2.2093x; 2.2093x over previous
import functools

import jax
import jax.numpy as jnp
from jax import lax
from jax.experimental import pallas as pl
from jax.experimental.pallas import tpu as pltpu
from jax.experimental.pallas import tpu_sc as plsc

B = 16384
INPUT_DIM = 128
N_CAT = 28
EMB = 16
OUT = 16

NC = 2
NS = 16
NW = NC * NS
B_PER_W = B // NW
CHUNK = 64
NCHUNK = B_PER_W // CHUNK

_sc_mesh = plsc.VectorSubcoreMesh(
    core_axis_name="c", subcore_axis_name="s", num_cores=NC, num_subcores=NS
)


def _sc_gather_sum_body(table_hbm, idx_hbm, out_hbm, idxb, rows, acc, sem):
    wid = lax.axis_index("s") * NC + lax.axis_index("c")
    row0 = wid * B_PER_W
    for ci in range(NCHUNK):
        off = (row0 + ci * CHUNK) * N_CAT
        pltpu.sync_copy(idx_hbm.at[pl.ds(off, CHUNK * N_CAT)], idxb)
        pltpu.async_copy(table_hbm.at[idxb], rows, sem).wait()

        def row_body(r, carry, ci=ci):
            base = r * N_CAT
            s = rows[base, :]
            for j in range(1, N_CAT):
                s = s + rows[base + j, :]
            acc[ci * CHUNK + r, :] = s
            return carry

        lax.fori_loop(0, CHUNK, row_body, 0)
    pltpu.sync_copy(acc, out_hbm.at[pl.ds(row0, B_PER_W)])


_SC_SCRATCH = [
    pltpu.VMEM((CHUNK * N_CAT,), jnp.int32),
    pltpu.VMEM((CHUNK * N_CAT, EMB), jnp.float32),
    pltpu.VMEM((B_PER_W, EMB), jnp.float32),
    pltpu.SemaphoreType.DMA,
]

_sc_gather_sum = pl.kernel(
    _sc_gather_sum_body,
    out_type=jax.ShapeDtypeStruct((B, EMB), jnp.float32),
    mesh=_sc_mesh,
    scratch_types=_SC_SCRATCH,
    compiler_params=pltpu.CompilerParams(use_tc_tiling_on_sc=False),
)


TB = 2048


def _mlp_body(feats_ref, w1p_ref, b1_ref, m2_ref, c_ref, out_ref):
    h = jnp.tanh(
        jnp.dot(feats_ref[:], w1p_ref[:], preferred_element_type=jnp.float32)
        + b1_ref[:]
    )
    out_ref[:] = (
        jnp.dot(h, m2_ref[:], preferred_element_type=jnp.float32) + c_ref[:]
    )


def _combine_body(part_ref, sums_ref, m1_ref, out_ref):
    out_ref[:] = part_ref[:] + jnp.dot(
        sums_ref[:], m1_ref[:], preferred_element_type=jnp.float32
    )


def kernel(ids, feats, table, w1, b1, w2, b2, wfc, bfc, wf, bf):
    del ids
    wfa = wf[:, :OUT]
    wfb = wf[:, OUT:]
    m1t = (wfa @ wfc).T / float(N_CAT)
    m2t = (wfb @ w2).T
    c = (bf + bfc @ wfa.T + b2 @ wfb.T).reshape(1, OUT)
    w1p = jnp.pad(w1, ((0, 0), (0, N_CAT))).T
    b1r = b1.reshape(1, OUT)

    cat_idx = feats[:, INPUT_DIM - N_CAT :].astype(jnp.int32).reshape(-1)

    sums = _sc_gather_sum(table, cat_idx)

    grid = (B // TB,)
    partial = pl.pallas_call(
        _mlp_body,
        grid=grid,
        in_specs=[
            pl.BlockSpec((TB, INPUT_DIM), lambda i: (i, 0)),
            pl.BlockSpec((INPUT_DIM, OUT), lambda i: (0, 0)),
            pl.BlockSpec((1, OUT), lambda i: (0, 0)),
            pl.BlockSpec((OUT, OUT), lambda i: (0, 0)),
            pl.BlockSpec((1, OUT), lambda i: (0, 0)),
        ],
        out_specs=pl.BlockSpec((TB, OUT), lambda i: (i, 0)),
        out_shape=jax.ShapeDtypeStruct((B, OUT), jnp.float32),
    )(feats, w1p, b1r, m2t, c)

    out = pl.pallas_call(
        _combine_body,
        grid=grid,
        in_specs=[
            pl.BlockSpec((TB, OUT), lambda i: (i, 0)),
            pl.BlockSpec((TB, EMB), lambda i: (i, 0)),
            pl.BlockSpec((EMB, OUT), lambda i: (0, 0)),
        ],
        out_specs=pl.BlockSpec((TB, OUT), lambda i: (i, 0)),
        out_shape=jax.ShapeDtypeStruct((B, OUT), jnp.float32),
    )(partial, sums, m1t)
    return out


# speedup vs baseline: 2.7266x; 1.2342x over previous
import functools

import jax
import jax.numpy as jnp
from jax import lax
from jax.experimental import pallas as pl
from jax.experimental.pallas import tpu as pltpu
from jax.experimental.pallas import tpu_sc as plsc

B = 16384
INPUT_DIM = 128
N_CAT = 28
VOCAB = 400000
EMB = 16
OUT = 16

NC = 2
NS = 16
NW = NC * NS
B_PER_W = B // NW
CHUNK = 64
NCHUNK = B_PER_W // CHUNK

def _sc_mesh():
    return plsc.VectorSubcoreMesh(
        core_axis_name="c", subcore_axis_name="s", num_cores=NC, num_subcores=NS
    )


def _sc_gather_sum_body(table_hbm, idx_hbm, out_hbm, idxb, rows, acc, sem):
    wid = lax.axis_index("s") * NC + lax.axis_index("c")
    row0 = wid * B_PER_W
    for ci in range(NCHUNK):
        off = (row0 + ci * CHUNK) * N_CAT
        pltpu.sync_copy(idx_hbm.at[pl.ds(off, CHUNK * N_CAT)], idxb)
        pltpu.async_copy(table_hbm.at[idxb], rows, sem).wait()

        def row_body(r, carry, ci=ci):
            base = r * N_CAT
            s = rows[base, :]
            for j in range(1, N_CAT):
                s = s + rows[base + j, :]
            acc[ci * CHUNK + r, :] = s
            return carry

        lax.fori_loop(0, CHUNK, row_body, 0)
    pltpu.sync_copy(acc, out_hbm.at[pl.ds(row0, B_PER_W)])


_SC_SCRATCH = [
    pltpu.VMEM((CHUNK * N_CAT,), jnp.int32),
    pltpu.VMEM((CHUNK * N_CAT, EMB), jnp.float32),
    pltpu.VMEM((B_PER_W, EMB), jnp.float32),
    pltpu.SemaphoreType.DMA,
]

_sc_cache = {}


def _sc_gather_sum(table2d, idx):
    if "k" not in _sc_cache:
        _sc_cache["k"] = pl.kernel(
            _sc_gather_sum_body,
            out_type=jax.ShapeDtypeStruct((B, EMB), jnp.float32),
            mesh=_sc_mesh(),
            scratch_types=_SC_SCRATCH,
            compiler_params=pltpu.CompilerParams(use_tc_tiling_on_sc=False),
        )
    return _sc_cache["k"](table2d, idx)


TB = 2048

NSTRIP = 8
VP_STRIP = 54400
RW = 3200
RG = VP_STRIP // RW
_LAST_BLK = VOCAB // RW - 1


def _repack_body(*refs):
    ins, out = refs[:NSTRIP], refs[NSTRIP]
    for s in range(NSTRIP):
        out[:, EMB * s : EMB * (s + 1)] = ins[s][:].T


def _repack_index_map(s):
    if s < NSTRIP - 1:
        return lambda k: (0, RG * s + k)
    return lambda k: (0, jnp.minimum(RG * s + k, _LAST_BLK))


def _mlp_body(feats_ref, w1p_ref, b1_ref, m2_ref, c_ref, out_ref):
    h = jnp.tanh(
        jnp.dot(feats_ref[:], w1p_ref[:], preferred_element_type=jnp.float32)
        + b1_ref[:]
    )
    out_ref[:] = (
        jnp.dot(h, m2_ref[:], preferred_element_type=jnp.float32) + c_ref[:]
    )


def _combine_body(part_ref, sums_ref, m1_ref, out_ref):
    out_ref[:] = part_ref[:] + jnp.dot(
        sums_ref[:], m1_ref[:], preferred_element_type=jnp.float32
    )


def kernel(ids, feats, table, w1, b1, w2, b2, wfc, bfc, wf, bf):
    del ids
    wfa = wf[:, :OUT]
    wfb = wf[:, OUT:]
    m1t = (wfa @ wfc).T / float(N_CAT)
    m2t = (wfb @ w2).T
    c = (bf + bfc @ wfa.T + b2 @ wfb.T).reshape(1, OUT)
    w1p = jnp.pad(w1, ((0, 0), (0, N_CAT))).T
    b1r = b1.reshape(1, OUT)

    cat_idx = feats[:, INPUT_DIM - N_CAT :].astype(jnp.int32).reshape(-1)
    strip = cat_idx // VP_STRIP
    cat_idx = NSTRIP * (cat_idx - VP_STRIP * strip) + strip

    t_lin = pl.pallas_call(
        _repack_body,
        grid=(RG,),
        in_specs=[pl.BlockSpec((EMB, RW), _repack_index_map(s)) for s in range(NSTRIP)],
        out_specs=pl.BlockSpec((RW, 128), lambda k: (k, 0)),
        out_shape=jax.ShapeDtypeStruct((VP_STRIP, 128), jnp.float32),
    )(*([table.T] * NSTRIP))

    sums = _sc_gather_sum(t_lin.reshape(VP_STRIP * NSTRIP, EMB), cat_idx)

    grid = (B // TB,)
    partial = pl.pallas_call(
        _mlp_body,
        grid=grid,
        in_specs=[
            pl.BlockSpec((TB, INPUT_DIM), lambda i: (i, 0)),
            pl.BlockSpec((INPUT_DIM, OUT), lambda i: (0, 0)),
            pl.BlockSpec((1, OUT), lambda i: (0, 0)),
            pl.BlockSpec((OUT, OUT), lambda i: (0, 0)),
            pl.BlockSpec((1, OUT), lambda i: (0, 0)),
        ],
        out_specs=pl.BlockSpec((TB, OUT), lambda i: (i, 0)),
        out_shape=jax.ShapeDtypeStruct((B, OUT), jnp.float32),
    )(feats, w1p, b1r, m2t, c)

    out = pl.pallas_call(
        _combine_body,
        grid=grid,
        in_specs=[
            pl.BlockSpec((TB, OUT), lambda i: (i, 0)),
            pl.BlockSpec((TB, EMB), lambda i: (i, 0)),
            pl.BlockSpec((EMB, OUT), lambda i: (0, 0)),
        ],
        out_specs=pl.BlockSpec((TB, OUT), lambda i: (i, 0)),
        out_shape=jax.ShapeDtypeStruct((B, OUT), jnp.float32),
    )(partial, sums, m1t)
    return out


# speedup vs baseline: 5.2406x; 1.9220x over previous
import functools

import jax
import jax.numpy as jnp
from jax import lax
from jax.experimental import pallas as pl
from jax.experimental.pallas import tpu as pltpu
from jax.experimental.pallas import tpu_sc as plsc

B = 16384
INPUT_DIM = 128
N_CAT = 28
VOCAB = 400000
EMB = 16
OUT = 16

NC = 2
NS = 16
NW = NC * NS
B_PER_W = B // NW
CHUNK = 64
NCHUNK = B_PER_W // CHUNK

def _sc_mesh():
    return plsc.VectorSubcoreMesh(
        core_axis_name="c", subcore_axis_name="s", num_cores=NC, num_subcores=NS
    )


def _sc_gather_sum_body(
    table_hbm, idx_hbm, out_hbm, idx0, idx1, rows0, rows1, acc, sem0, sem1
):
    wid = lax.axis_index("s") * NC + lax.axis_index("c")
    row0 = wid * B_PER_W
    idxb = (idx0, idx1)
    rows = (rows0, rows1)
    sems = (sem0, sem1)
    CW = CHUNK * N_CAT

    pltpu.sync_copy(idx_hbm.at[pl.ds(row0 * N_CAT, CW)], idx0)
    copies = [pltpu.async_copy(table_hbm.at[idx0], rows0, sem0), None]
    for ci in range(NCHUNK):
        p = ci % 2
        q = 1 - p
        if ci + 1 < NCHUNK:
            off = (row0 + (ci + 1) * CHUNK) * N_CAT
            pltpu.sync_copy(idx_hbm.at[pl.ds(off, CW)], idxb[q])
            copies[q] = pltpu.async_copy(table_hbm.at[idxb[q]], rows[q], sems[q])
        copies[p].wait()
        r = rows[p]

        def row_body(rr, carry, ci=ci, r=r):
            base = rr * N_CAT
            s = r[base, :]
            for j in range(1, N_CAT):
                s = s + r[base + j, :]
            acc[ci * CHUNK + rr, :] = s
            return carry

        lax.fori_loop(0, CHUNK, row_body, 0)
    pltpu.sync_copy(acc, out_hbm.at[pl.ds(row0, B_PER_W)])


_SC_SCRATCH = [
    pltpu.VMEM((CHUNK * N_CAT,), jnp.int32),
    pltpu.VMEM((CHUNK * N_CAT,), jnp.int32),
    pltpu.VMEM((CHUNK * N_CAT, EMB), jnp.float32),
    pltpu.VMEM((CHUNK * N_CAT, EMB), jnp.float32),
    pltpu.VMEM((B_PER_W, EMB), jnp.float32),
    pltpu.SemaphoreType.DMA,
    pltpu.SemaphoreType.DMA,
]

_sc_cache = {}


def _sc_gather_sum(table2d, idx):
    if "k" not in _sc_cache:
        _sc_cache["k"] = pl.kernel(
            _sc_gather_sum_body,
            out_type=jax.ShapeDtypeStruct((B, EMB), jnp.float32),
            mesh=_sc_mesh(),
            scratch_types=_SC_SCRATCH,
            compiler_params=pltpu.CompilerParams(use_tc_tiling_on_sc=False),
        )
    return _sc_cache["k"](table2d, idx)


TB = 2048

NSTRIP = 8
VP_STRIP = 54400
RW = 3200
RG = VP_STRIP // RW
_LAST_BLK = VOCAB // RW - 1


def _repack_body(*refs):
    ins, out = refs[:NSTRIP], refs[NSTRIP]
    out[:] = jnp.concatenate([ins[s][:] for s in range(NSTRIP)], axis=0).T


def _repack_index_map(s):
    if s < NSTRIP - 1:
        return lambda k: (0, RG * s + k)
    return lambda k: (0, jnp.minimum(RG * s + k, _LAST_BLK))


def _mlp_body(feats_ref, w1p_ref, b1_ref, m2_ref, c_ref, out_ref):
    h = jnp.tanh(
        jnp.dot(feats_ref[:], w1p_ref[:], preferred_element_type=jnp.float32)
        + b1_ref[:]
    )
    out_ref[:] = (
        jnp.dot(h, m2_ref[:], preferred_element_type=jnp.float32) + c_ref[:]
    )


def _combine_body(part_ref, sums_ref, m1_ref, out_ref):
    out_ref[:] = part_ref[:] + jnp.dot(
        sums_ref[:], m1_ref[:], preferred_element_type=jnp.float32
    )


def kernel(ids, feats, table, w1, b1, w2, b2, wfc, bfc, wf, bf):
    del ids
    wfa = wf[:, :OUT]
    wfb = wf[:, OUT:]
    m1t = (wfa @ wfc).T / float(N_CAT)
    m2t = (wfb @ w2).T
    c = (bf + bfc @ wfa.T + b2 @ wfb.T).reshape(1, OUT)
    w1p = jnp.pad(w1, ((0, 0), (0, N_CAT))).T
    b1r = b1.reshape(1, OUT)

    cat_idx = feats[:, INPUT_DIM - N_CAT :].astype(jnp.int32).reshape(-1)
    strip = cat_idx // VP_STRIP
    cat_idx = NSTRIP * (cat_idx - VP_STRIP * strip) + strip

    t_lin = pl.pallas_call(
        _repack_body,
        grid=(RG,),
        in_specs=[pl.BlockSpec((EMB, RW), _repack_index_map(s)) for s in range(NSTRIP)],
        out_specs=pl.BlockSpec((RW, 128), lambda k: (k, 0)),
        out_shape=jax.ShapeDtypeStruct((VP_STRIP, 128), jnp.float32),
    )(*([table.T] * NSTRIP))

    sums = _sc_gather_sum(t_lin.reshape(VP_STRIP * NSTRIP, EMB), cat_idx)

    grid = (B // TB,)
    partial = pl.pallas_call(
        _mlp_body,
        grid=grid,
        in_specs=[
            pl.BlockSpec((TB, INPUT_DIM), lambda i: (i, 0)),
            pl.BlockSpec((INPUT_DIM, OUT), lambda i: (0, 0)),
            pl.BlockSpec((1, OUT), lambda i: (0, 0)),
            pl.BlockSpec((OUT, OUT), lambda i: (0, 0)),
            pl.BlockSpec((1, OUT), lambda i: (0, 0)),
        ],
        out_specs=pl.BlockSpec((TB, OUT), lambda i: (i, 0)),
        out_shape=jax.ShapeDtypeStruct((B, OUT), jnp.float32),
    )(feats, w1p, b1r, m2t, c)

    out = pl.pallas_call(
        _combine_body,
        grid=grid,
        in_specs=[
            pl.BlockSpec((TB, OUT), lambda i: (i, 0)),
            pl.BlockSpec((TB, EMB), lambda i: (i, 0)),
            pl.BlockSpec((EMB, OUT), lambda i: (0, 0)),
        ],
        out_specs=pl.BlockSpec((TB, OUT), lambda i: (i, 0)),
        out_shape=jax.ShapeDtypeStruct((B, OUT), jnp.float32),
    )(partial, sums, m1t)
    return out


# speedup vs baseline: 6.2183x; 1.1866x over previous
import functools

import jax
import jax.numpy as jnp
from jax import lax
from jax.experimental import pallas as pl
from jax.experimental.pallas import tpu as pltpu
from jax.experimental.pallas import tpu_sc as plsc

B = 16384
INPUT_DIM = 128
N_CAT = 28
VOCAB = 400000
EMB = 16
OUT = 16

NSTRIP = 8
VP_STRIP = 54400
RW = 3200
RG = VP_STRIP // RW
_LAST_BLK = VOCAB // RW - 1

NC = 2
NS = 16
NW = NC * NS
B_PER_W = B // NW
CHUNK = 64
NCHUNK = B_PER_W // CHUNK

def _sc_mesh():
    return plsc.VectorSubcoreMesh(
        core_axis_name="c", subcore_axis_name="s", num_cores=NC, num_subcores=NS
    )


_INV_STRIP = 1.0 / VP_STRIP


def _build_idx(featsb, idxb):
    def one_row(rr, carry):
        fbase = rr * INPUT_DIM + (INPUT_DIM - N_CAT)
        ibase = rr * N_CAT
        for fo, io in ((0, 0), (N_CAT - EMB, N_CAT - EMB)):
            v = featsb[pl.ds(fbase + fo, EMB)]
            s = ((v + 0.5) * _INV_STRIP).astype(jnp.int32)
            u = v.astype(jnp.int32) - VP_STRIP * s
            idxb[pl.ds(ibase + io, EMB)] = NSTRIP * u + s
        return carry

    lax.fori_loop(0, CHUNK, one_row, 0)


def _sc_gather_sum_body(
    table_hbm, feats_hbm, out_hbm, f0, f1, idx0, idx1, rows0, rows1, acc,
    fsem0, fsem1, sem0, sem1
):
    wid = lax.axis_index("s") * NC + lax.axis_index("c")
    row0 = wid * B_PER_W
    fb = (f0, f1)
    idxb = (idx0, idx1)
    rows = (rows0, rows1)
    fsems = (fsem0, fsem1)
    sems = (sem0, sem1)
    FW = CHUNK * INPUT_DIM

    def feats_copy(ci, p):
        return pltpu.async_copy(
            feats_hbm.at[pl.ds((row0 + ci * CHUNK) * INPUT_DIM, FW)], fb[p], fsems[p]
        )

    feats_copy(0, 0).wait()
    _build_idx(f0, idx0)
    copies = [pltpu.async_copy(table_hbm.at[idx0], rows0, sem0), None]
    fcopies = [None, feats_copy(1, 1) if NCHUNK > 1 else None]
    for ci in range(NCHUNK):
        p = ci % 2
        q = 1 - p
        if ci + 1 < NCHUNK:
            fcopies[q].wait()
            _build_idx(fb[q], idxb[q])
            copies[q] = pltpu.async_copy(table_hbm.at[idxb[q]], rows[q], sems[q])
            if ci + 2 < NCHUNK:
                fcopies[p] = feats_copy(ci + 2, p)
        copies[p].wait()
        r = rows[p]

        def row_body(rr, carry, ci=ci, r=r):
            base = rr * N_CAT
            vs = [r[base + j, :] for j in range(N_CAT)]
            while len(vs) > 1:
                nxt = [vs[k] + vs[k + 1] for k in range(0, len(vs) - 1, 2)]
                if len(vs) % 2:
                    nxt.append(vs[-1])
                vs = nxt
            acc[ci * CHUNK + rr, :] = vs[0]
            return carry

        lax.fori_loop(0, CHUNK, row_body, 0)
    pltpu.sync_copy(acc, out_hbm.at[pl.ds(row0, B_PER_W)])


_SC_SCRATCH = [
    pltpu.VMEM((CHUNK * INPUT_DIM,), jnp.float32),
    pltpu.VMEM((CHUNK * INPUT_DIM,), jnp.float32),
    pltpu.VMEM((CHUNK * N_CAT,), jnp.int32),
    pltpu.VMEM((CHUNK * N_CAT,), jnp.int32),
    pltpu.VMEM((CHUNK * N_CAT, EMB), jnp.float32),
    pltpu.VMEM((CHUNK * N_CAT, EMB), jnp.float32),
    pltpu.VMEM((B_PER_W, EMB), jnp.float32),
    pltpu.SemaphoreType.DMA,
    pltpu.SemaphoreType.DMA,
    pltpu.SemaphoreType.DMA,
    pltpu.SemaphoreType.DMA,
]

_sc_cache = {}


def _sc_gather_sum(table2d, feats_flat):
    if "k" not in _sc_cache:
        _sc_cache["k"] = pl.kernel(
            _sc_gather_sum_body,
            out_type=jax.ShapeDtypeStruct((B, EMB), jnp.float32),
            mesh=_sc_mesh(),
            scratch_types=_SC_SCRATCH,
            compiler_params=pltpu.CompilerParams(use_tc_tiling_on_sc=False),
        )
    return _sc_cache["k"](table2d, feats_flat)


TB = 2048

def _repack_body(*refs):
    ins, out = refs[:NSTRIP], refs[NSTRIP]
    out[:] = jnp.concatenate([ins[s][:] for s in range(NSTRIP)], axis=0).T


def _repack_index_map(s):
    if s < NSTRIP - 1:
        return lambda k: (0, RG * s + k)
    return lambda k: (0, jnp.minimum(RG * s + k, _LAST_BLK))


def _mlp_body(feats_ref, w1p_ref, b1_ref, m2_ref, c_ref, out_ref):
    h = jnp.tanh(
        jnp.dot(feats_ref[:], w1p_ref[:], preferred_element_type=jnp.float32)
        + b1_ref[:]
    )
    out_ref[:] = (
        jnp.dot(h, m2_ref[:], preferred_element_type=jnp.float32) + c_ref[:]
    )


def _combine_body(part_ref, sums_ref, m1_ref, out_ref):
    out_ref[:] = part_ref[:] + jnp.dot(
        sums_ref[:], m1_ref[:], preferred_element_type=jnp.float32
    )


def kernel(ids, feats, table, w1, b1, w2, b2, wfc, bfc, wf, bf):
    del ids
    wfa = wf[:, :OUT]
    wfb = wf[:, OUT:]
    m1t = (wfa @ wfc).T / float(N_CAT)
    m2t = (wfb @ w2).T
    c = (bf + bfc @ wfa.T + b2 @ wfb.T).reshape(1, OUT)
    w1p = jnp.pad(w1, ((0, 0), (0, N_CAT))).T
    b1r = b1.reshape(1, OUT)

    t_lin = pl.pallas_call(
        _repack_body,
        grid=(RG,),
        in_specs=[pl.BlockSpec((EMB, RW), _repack_index_map(s)) for s in range(NSTRIP)],
        out_specs=pl.BlockSpec((RW, 128), lambda k: (k, 0)),
        out_shape=jax.ShapeDtypeStruct((VP_STRIP, 128), jnp.float32),
    )(*([table.T] * NSTRIP))

    sums = _sc_gather_sum(t_lin.reshape(VP_STRIP * NSTRIP, EMB), feats.reshape(-1))

    grid = (B // TB,)
    partial = pl.pallas_call(
        _mlp_body,
        grid=grid,
        in_specs=[
            pl.BlockSpec((TB, INPUT_DIM), lambda i: (i, 0)),
            pl.BlockSpec((INPUT_DIM, OUT), lambda i: (0, 0)),
            pl.BlockSpec((1, OUT), lambda i: (0, 0)),
            pl.BlockSpec((OUT, OUT), lambda i: (0, 0)),
            pl.BlockSpec((1, OUT), lambda i: (0, 0)),
        ],
        out_specs=pl.BlockSpec((TB, OUT), lambda i: (i, 0)),
        out_shape=jax.ShapeDtypeStruct((B, OUT), jnp.float32),
    )(feats, w1p, b1r, m2t, c)

    out = pl.pallas_call(
        _combine_body,
        grid=grid,
        in_specs=[
            pl.BlockSpec((TB, OUT), lambda i: (i, 0)),
            pl.BlockSpec((TB, EMB), lambda i: (i, 0)),
            pl.BlockSpec((EMB, OUT), lambda i: (0, 0)),
        ],
        out_specs=pl.BlockSpec((TB, OUT), lambda i: (i, 0)),
        out_shape=jax.ShapeDtypeStruct((B, OUT), jnp.float32),
    )(partial, sums, m1t)
    return out


# speedup vs baseline: 7.2115x; 1.1597x over previous
import functools

import jax
import jax.numpy as jnp
from jax import lax
from jax.experimental import pallas as pl
from jax.experimental.pallas import tpu as pltpu
from jax.experimental.pallas import tpu_sc as plsc

B = 16384
INPUT_DIM = 128
N_CAT = 28
VOCAB = 400000
EMB = 16
OUT = 16

NSTRIP = 8
VP_STRIP = 54400
RW = 3200
RG = VP_STRIP // RW
_LAST_BLK = VOCAB // RW - 1

NC = 2
NS = 16
NW = NC * NS
B_PER_W = B // NW
CHUNK = 64
NCHUNK = B_PER_W // CHUNK

def _sc_mesh():
    return plsc.VectorSubcoreMesh(
        core_axis_name="c", subcore_axis_name="s", num_cores=NC, num_subcores=NS
    )


_INV_STRIP = 1.0 / VP_STRIP


def _build_idx(featsb, idxb):
    def one_row(rr, carry):
        fbase = rr * INPUT_DIM + (INPUT_DIM - N_CAT)
        ibase = rr * N_CAT
        for fo, io in ((0, 0), (N_CAT - EMB, N_CAT - EMB)):
            v = featsb[pl.ds(fbase + fo, EMB)]
            s = ((v + 0.5) * _INV_STRIP).astype(jnp.int32)
            u = v.astype(jnp.int32) - VP_STRIP * s
            idxb[pl.ds(ibase + io, EMB)] = NSTRIP * u + s
        return carry

    lax.fori_loop(0, CHUNK, one_row, 0)


def _sc_gather_sum_body(
    table_hbm, feats_hbm, out_hbm, f0, f1, idx0, idx1, rows0, rows1, acc,
    fsem0, fsem1, sem0, sem1
):
    wid = lax.axis_index("s") * NC + lax.axis_index("c")
    row0 = wid * B_PER_W
    fb = (f0, f1)
    idxb = (idx0, idx1)
    rows = (rows0, rows1)
    fsems = (fsem0, fsem1)
    sems = (sem0, sem1)
    FW = CHUNK * INPUT_DIM

    def feats_copy(ci, p):
        return pltpu.async_copy(
            feats_hbm.at[pl.ds((row0 + ci * CHUNK) * INPUT_DIM, FW)], fb[p], fsems[p]
        )

    feats_copy(0, 0).wait()
    _build_idx(f0, idx0)
    copies = [pltpu.async_copy(table_hbm.at[idx0], rows0, sem0), None]
    fcopies = [None, feats_copy(1, 1) if NCHUNK > 1 else None]
    for ci in range(NCHUNK):
        p = ci % 2
        q = 1 - p
        if ci + 1 < NCHUNK:
            fcopies[q].wait()
            _build_idx(fb[q], idxb[q])
            copies[q] = pltpu.async_copy(table_hbm.at[idxb[q]], rows[q], sems[q])
            if ci + 2 < NCHUNK:
                fcopies[p] = feats_copy(ci + 2, p)
        copies[p].wait()
        r = rows[p]
        rowid = lax.iota(jnp.int32, EMB)

        def row_body(rr, carry, ci=ci, r=r):
            base = rr * N_CAT
            vs = [r[base + j, :] for j in range(N_CAT)]
            while len(vs) > 1:
                nxt = [vs[k] + vs[k + 1] for k in range(0, len(vs) - 1, 2)]
                if len(vs) % 2:
                    nxt.append(vs[-1])
                vs = nxt
            col = jnp.full((EMB,), ci * CHUNK, jnp.int32) + rr
            plsc.store_scatter(acc, [rowid, col], vs[0])
            return carry

        lax.fori_loop(0, CHUNK, row_body, 0)
    pltpu.sync_copy(acc, out_hbm.at[:, pl.ds(row0, B_PER_W)])


_SC_SCRATCH = [
    pltpu.VMEM((CHUNK * INPUT_DIM,), jnp.float32),
    pltpu.VMEM((CHUNK * INPUT_DIM,), jnp.float32),
    pltpu.VMEM((CHUNK * N_CAT,), jnp.int32),
    pltpu.VMEM((CHUNK * N_CAT,), jnp.int32),
    pltpu.VMEM((CHUNK * N_CAT, EMB), jnp.float32),
    pltpu.VMEM((CHUNK * N_CAT, EMB), jnp.float32),
    pltpu.VMEM((EMB, B_PER_W), jnp.float32),
    pltpu.SemaphoreType.DMA,
    pltpu.SemaphoreType.DMA,
    pltpu.SemaphoreType.DMA,
    pltpu.SemaphoreType.DMA,
]

_sc_cache = {}


def _sc_gather_sum(table2d, feats_flat):
    if "k" not in _sc_cache:
        _sc_cache["k"] = pl.kernel(
            _sc_gather_sum_body,
            out_type=jax.ShapeDtypeStruct((EMB, B), jnp.float32),
            mesh=_sc_mesh(),
            scratch_types=_SC_SCRATCH,
            compiler_params=pltpu.CompilerParams(
                use_tc_tiling_on_sc=False, needs_layout_passes=False
            ),
        )
    return _sc_cache["k"](table2d, feats_flat)


TB = 2048

def _repack_body(*refs):
    ins, out = refs[:NSTRIP], refs[NSTRIP]
    out[:] = jnp.concatenate([ins[s][:] for s in range(NSTRIP)], axis=0).T


def _repack_index_map(s):
    if s < NSTRIP - 1:
        return lambda k: (0, RG * s + k)
    return lambda k: (0, jnp.minimum(RG * s + k, _LAST_BLK))


def _mlp_body(feats_ref, w1p_ref, b1_ref, m2t_ref, c_ref, out_ref):
    ht = jnp.tanh(
        jax.lax.dot_general(
            w1p_ref[:], feats_ref[:], (((1,), (1,)), ((), ())),
            preferred_element_type=jnp.float32,
        )
        + b1_ref[:]
    )
    out_ref[:] = (
        jax.lax.dot_general(
            m2t_ref[:], ht, (((0,), (0,)), ((), ())),
            preferred_element_type=jnp.float32,
        )
        + c_ref[:]
    )


def _combine_body(part_ref, sums_ref, m1t_ref, out_ref):
    out_ref[:] = part_ref[:] + jax.lax.dot_general(
        m1t_ref[:], sums_ref[:], (((0,), (0,)), ((), ())),
        preferred_element_type=jnp.float32,
    )


def kernel(ids, feats, table, w1, b1, w2, b2, wfc, bfc, wf, bf):
    del ids
    wfa = wf[:, :OUT]
    wfb = wf[:, OUT:]
    m1t = (wfa @ wfc).T / float(N_CAT)
    m2t = (wfb @ w2).T
    c = (bf + bfc @ wfa.T + b2 @ wfb.T).reshape(OUT, 1)
    w1p = jnp.pad(w1, ((0, 0), (0, N_CAT)))
    b1r = b1.reshape(OUT, 1)

    t_lin = pl.pallas_call(
        _repack_body,
        grid=(RG,),
        in_specs=[pl.BlockSpec((EMB, RW), _repack_index_map(s)) for s in range(NSTRIP)],
        out_specs=pl.BlockSpec((RW, 128), lambda k: (k, 0)),
        out_shape=jax.ShapeDtypeStruct((VP_STRIP, 128), jnp.float32),
    )(*([table.T] * NSTRIP))

    sums = _sc_gather_sum(t_lin.reshape(VP_STRIP * NSTRIP, EMB), feats.reshape(-1))

    grid = (B // TB,)
    partial_t = pl.pallas_call(
        _mlp_body,
        grid=grid,
        in_specs=[
            pl.BlockSpec((TB, INPUT_DIM), lambda i: (i, 0)),
            pl.BlockSpec((OUT, INPUT_DIM), lambda i: (0, 0)),
            pl.BlockSpec((OUT, 1), lambda i: (0, 0)),
            pl.BlockSpec((OUT, OUT), lambda i: (0, 0)),
            pl.BlockSpec((OUT, 1), lambda i: (0, 0)),
        ],
        out_specs=pl.BlockSpec((OUT, TB), lambda i: (0, i)),
        out_shape=jax.ShapeDtypeStruct((OUT, B), jnp.float32),
    )(feats, w1p, b1r, m2t, c)

    out_t = pl.pallas_call(
        _combine_body,
        grid=grid,
        in_specs=[
            pl.BlockSpec((OUT, TB), lambda i: (0, i)),
            pl.BlockSpec((EMB, TB), lambda i: (0, i)),
            pl.BlockSpec((EMB, OUT), lambda i: (0, 0)),
        ],
        out_specs=pl.BlockSpec((OUT, TB), lambda i: (0, i)),
        out_shape=jax.ShapeDtypeStruct((OUT, B), jnp.float32),
    )(partial_t, sums, m1t)
    return out_t.T


# speedup vs baseline: 8.1543x; 1.1307x over previous
import functools

import jax
import jax.numpy as jnp
from jax import lax
from jax.experimental import pallas as pl
from jax.experimental.pallas import tpu as pltpu
from jax.experimental.pallas import tpu_sc as plsc

B = 16384
INPUT_DIM = 128
N_CAT = 28
VOCAB = 400000
EMB = 16
OUT = 16

NSTRIP = 8
VP_STRIP = 54400
RW = 3200
RG = VP_STRIP // RW
_LAST_BLK = VOCAB // RW - 1

NC = 2
NS = 16
NW = NC * NS
B_PER_W = B // NW
CHUNK = 64
NCHUNK = B_PER_W // CHUNK

def _sc_mesh():
    return plsc.VectorSubcoreMesh(
        core_axis_name="c", subcore_axis_name="s", num_cores=NC, num_subcores=NS
    )


_INV_STRIP = 1.0 / VP_STRIP


def _build_idx(featsb, idxb):
    @plsc.parallel_loop(0, CHUNK, unroll=2)
    def one_row(rr):
        fbase = rr * INPUT_DIM + (INPUT_DIM - N_CAT)
        ibase = rr * N_CAT
        for fo, io in ((0, 0), (N_CAT - EMB, N_CAT - EMB)):
            v = featsb[pl.ds(fbase + fo, EMB)]
            s = ((v + 0.5) * _INV_STRIP).astype(jnp.int32)
            u = v.astype(jnp.int32) - VP_STRIP * s
            idxb[pl.ds(ibase + io, EMB)] = NSTRIP * u + s


def _sc_gather_sum_body(
    table_hbm, feats_hbm, out_hbm, f0, f1, idx0, idx1, rows0, rows1, acc,
    fsem0, fsem1, sem0, sem1
):
    wid = lax.axis_index("s") * NC + lax.axis_index("c")
    row0 = wid * B_PER_W
    fb = (f0, f1)
    idxb = (idx0, idx1)
    rows = (rows0, rows1)
    fsems = (fsem0, fsem1)
    sems = (sem0, sem1)
    FW = CHUNK * INPUT_DIM

    def feats_copy(ci, p):
        return pltpu.async_copy(
            feats_hbm.at[pl.ds((row0 + ci * CHUNK) * INPUT_DIM, FW)], fb[p], fsems[p]
        )

    feats_copy(0, 0).wait()
    _build_idx(f0, idx0)
    copies = [pltpu.async_copy(table_hbm.at[idx0], rows0, sem0), None]
    fcopies = [None, feats_copy(1, 1) if NCHUNK > 1 else None]
    for ci in range(NCHUNK):
        p = ci % 2
        q = 1 - p
        if ci + 1 < NCHUNK:
            fcopies[q].wait()
            _build_idx(fb[q], idxb[q])
            copies[q] = pltpu.async_copy(table_hbm.at[idxb[q]], rows[q], sems[q])
            if ci + 2 < NCHUNK:
                fcopies[p] = feats_copy(ci + 2, p)
        copies[p].wait()
        r = rows[p]
        rowid = lax.iota(jnp.int32, EMB)

        @plsc.parallel_loop(0, CHUNK, unroll=2)
        def row_body(rr, ci=ci, r=r):
            base = rr * N_CAT
            vs = [r[base + j, :] for j in range(N_CAT)]
            while len(vs) > 1:
                nxt = [vs[k] + vs[k + 1] for k in range(0, len(vs) - 1, 2)]
                if len(vs) % 2:
                    nxt.append(vs[-1])
                vs = nxt
            col = jnp.full((EMB,), ci * CHUNK, jnp.int32) + rr
            plsc.store_scatter(acc, [rowid, col], vs[0])
    pltpu.sync_copy(acc, out_hbm.at[:, pl.ds(row0, B_PER_W)])


_SC_SCRATCH = [
    pltpu.VMEM((CHUNK * INPUT_DIM,), jnp.float32),
    pltpu.VMEM((CHUNK * INPUT_DIM,), jnp.float32),
    pltpu.VMEM((CHUNK * N_CAT,), jnp.int32),
    pltpu.VMEM((CHUNK * N_CAT,), jnp.int32),
    pltpu.VMEM((CHUNK * N_CAT, EMB), jnp.float32),
    pltpu.VMEM((CHUNK * N_CAT, EMB), jnp.float32),
    pltpu.VMEM((EMB, B_PER_W), jnp.float32),
    pltpu.SemaphoreType.DMA,
    pltpu.SemaphoreType.DMA,
    pltpu.SemaphoreType.DMA,
    pltpu.SemaphoreType.DMA,
]

_sc_cache = {}


def _sc_gather_sum(table2d, feats_flat):
    if "k" not in _sc_cache:
        _sc_cache["k"] = pl.kernel(
            _sc_gather_sum_body,
            out_type=jax.ShapeDtypeStruct((EMB, B), jnp.float32),
            mesh=_sc_mesh(),
            scratch_types=_SC_SCRATCH,
            compiler_params=pltpu.CompilerParams(
                use_tc_tiling_on_sc=False, needs_layout_passes=False
            ),
        )
    return _sc_cache["k"](table2d, feats_flat)


TB = 2048

def _repack_body(*refs):
    ins, out = refs[:NSTRIP], refs[NSTRIP]
    out[:] = jnp.concatenate([ins[s][:] for s in range(NSTRIP)], axis=0).T


def _repack_index_map(s):
    if s < NSTRIP - 1:
        return lambda k: (0, RG * s + k)
    return lambda k: (0, jnp.minimum(RG * s + k, _LAST_BLK))


def _mlp_body(feats_ref, w1p_ref, b1_ref, m2t_ref, c_ref, out_ref):
    ht = jnp.tanh(
        jax.lax.dot_general(
            w1p_ref[:], feats_ref[:], (((1,), (1,)), ((), ())),
            preferred_element_type=jnp.float32,
        )
        + b1_ref[:]
    )
    out_ref[:] = (
        jax.lax.dot_general(
            m2t_ref[:], ht, (((0,), (0,)), ((), ())),
            preferred_element_type=jnp.float32,
        )
        + c_ref[:]
    )


def _combine_body(part_ref, sums_ref, m1t_ref, out_ref):
    out_ref[:] = part_ref[:] + jax.lax.dot_general(
        m1t_ref[:], sums_ref[:], (((0,), (0,)), ((), ())),
        preferred_element_type=jnp.float32,
    )


def kernel(ids, feats, table, w1, b1, w2, b2, wfc, bfc, wf, bf):
    del ids
    wfa = wf[:, :OUT]
    wfb = wf[:, OUT:]
    m1t = (wfa @ wfc).T / float(N_CAT)
    m2t = (wfb @ w2).T
    c = (bf + bfc @ wfa.T + b2 @ wfb.T).reshape(OUT, 1)
    w1p = jnp.pad(w1, ((0, 0), (0, N_CAT)))
    b1r = b1.reshape(OUT, 1)

    t_lin = pl.pallas_call(
        _repack_body,
        grid=(RG,),
        in_specs=[pl.BlockSpec((EMB, RW), _repack_index_map(s)) for s in range(NSTRIP)],
        out_specs=pl.BlockSpec((RW, 128), lambda k: (k, 0)),
        out_shape=jax.ShapeDtypeStruct((VP_STRIP, 128), jnp.float32),
    )(*([table.T] * NSTRIP))

    sums = _sc_gather_sum(t_lin.reshape(VP_STRIP * NSTRIP, EMB), feats.reshape(-1))

    grid = (B // TB,)
    partial_t = pl.pallas_call(
        _mlp_body,
        grid=grid,
        in_specs=[
            pl.BlockSpec((TB, INPUT_DIM), lambda i: (i, 0)),
            pl.BlockSpec((OUT, INPUT_DIM), lambda i: (0, 0)),
            pl.BlockSpec((OUT, 1), lambda i: (0, 0)),
            pl.BlockSpec((OUT, OUT), lambda i: (0, 0)),
            pl.BlockSpec((OUT, 1), lambda i: (0, 0)),
        ],
        out_specs=pl.BlockSpec((OUT, TB), lambda i: (0, i)),
        out_shape=jax.ShapeDtypeStruct((OUT, B), jnp.float32),
    )(feats, w1p, b1r, m2t, c)

    TBC = 4096
    out_t = pl.pallas_call(
        _combine_body,
        grid=(B // TBC,),
        in_specs=[
            pl.BlockSpec((OUT, TBC), lambda i: (0, i)),
            pl.BlockSpec((EMB, TBC), lambda i: (0, i)),
            pl.BlockSpec((EMB, OUT), lambda i: (0, 0)),
        ],
        out_specs=pl.BlockSpec((OUT, TBC), lambda i: (0, i)),
        out_shape=jax.ShapeDtypeStruct((OUT, B), jnp.float32),
    )(partial_t, sums, m1t)
    return out_t.T
